# baseline (speedup 1.0000x reference)
import functools

import jax
import jax.numpy as jnp
from jax.experimental import pallas as pl
from jax.experimental.pallas import tpu as pltpu

_K = 1024
_D = 64
_TB = 1024
_COMMIT = 0.25


def _colsum_body(wt_ref, cs_ref):
    sq = wt_ref[...] * wt_ref[...]
    acc = sq[0:8, :]
    for j in range(1, 8):
        acc = acc + sq[8 * j:8 * j + 8, :]
    a4 = acc[0:4, :] + acc[4:8, :]
    a2 = a4[0:2, :] + a4[2:4, :]
    cs_ref[...] = a2[0:1, :] + a2[1:2, :]


def _vq_body(x_ref, w_ref, w2_ref, cs_ref, qst_ref, idx_ref, esq_ref,
             counts_ref):
    xb = x_ref[...]
    w = w_ref[...]
    sq = xb * xb
    r64 = jax.lax.broadcasted_iota(jnp.int32, (_D, 8), 0)
    c8 = jax.lax.broadcasted_iota(jnp.int32, (_D, 8), 1)
    sel = (jax.lax.rem(r64, 8) == c8).astype(jnp.float32)
    p = jax.lax.dot_general(sq, sel, (((1,), (0,)), ((), ())),
                            preferred_element_type=jnp.float32)
    a4 = p[:, 0:4] + p[:, 4:8]
    a2 = a4[:, 0:2] + a4[:, 2:4]
    rowsum = a2[:, 0:1] + a2[:, 1:2]
    colsum = cs_ref[...]
    mm2 = jax.lax.dot_general(xb, w2_ref[...], (((1,), (1,)), ((), ())),
                              preferred_element_type=jnp.float32)
    d = (rowsum + colsum) - mm2
    d = jnp.maximum(d, 0.0)
    dmin = jnp.min(d, axis=1, keepdims=True)
    iota = jax.lax.broadcasted_iota(jnp.int32, (_TB, _K), 1)
    idx = jnp.min(jnp.where(d == dmin, iota, _K), axis=1).astype(jnp.int32)

    onehot = (iota == idx[:, None]).astype(jnp.float32).astype(jnp.bfloat16)
    q = jax.lax.dot_general(onehot, w.astype(jnp.bfloat16),
                            (((1,), (0,)), ((), ())),
                            preferred_element_type=jnp.float32)
    e = q - xb
    qst_ref[...] = xb + e
    idx_ref[...] = idx[:, None]
    esq = e * e
    ones_d = jnp.ones((1, _D), jnp.float32)
    s1 = jax.lax.dot_general(esq, ones_d, (((1,), (1,)), ((), ())),
                             preferred_element_type=jnp.float32)
    ones_row = jnp.ones((1, _TB), jnp.float32)
    esq_ref[...] = jax.lax.dot_general(
        ones_row, s1, (((1,), (0,)), ((), ())),
        preferred_element_type=jnp.float32)[None]
    counts_ref[...] = jax.lax.dot_general(
        ones_row.astype(jnp.bfloat16), onehot, (((1,), (0,)), ((), ())),
        preferred_element_type=jnp.float32)[None]


def _fin_body(esq_ref, cp_ref, loss_ref, perp_ref, *, total_tokens):
    ssum = jnp.sum(esq_ref[...])
    loss = ssum / (total_tokens * _D)
    loss_ref[...] = jnp.full((1, 1), loss + _COMMIT * loss, jnp.float32)
    counts = jnp.sum(cp_ref[...], axis=0)
    p = counts * (1.0 / total_tokens)
    ent = jnp.sum(p * jnp.log(p + 1e-10))
    perp_ref[...] = jnp.full((1, 1), jnp.exp(-ent), jnp.float32)


@jax.jit
def kernel(x, W):
    orig_shape = x.shape
    x_flat = x.reshape(-1, _D)
    total = x_flat.shape[0]
    grid_n = total // _TB

    colsum = pl.pallas_call(
        _colsum_body,
        in_specs=[pl.BlockSpec((_D, _K), lambda: (0, 0))],
        out_specs=pl.BlockSpec((1, _K), lambda: (0, 0)),
        out_shape=jax.ShapeDtypeStruct((1, _K), jnp.float32),
    )(W.T)

    qst, idx2, esqs, counts_p = pl.pallas_call(
        _vq_body,
        grid=(grid_n,),
        in_specs=[
            pl.BlockSpec((_TB, _D), lambda i: (i, 0)),
            pl.BlockSpec((_K, _D), lambda i: (0, 0)),
            pl.BlockSpec((_K, _D), lambda i: (0, 0)),
            pl.BlockSpec((1, _K), lambda i: (0, 0)),
        ],
        out_specs=[
            pl.BlockSpec((_TB, _D), lambda i: (i, 0)),
            pl.BlockSpec((_TB, 1), lambda i: (i, 0)),
            pl.BlockSpec((1, 1, 1), lambda i: (i, 0, 0)),
            pl.BlockSpec((1, 1, _K), lambda i: (i, 0, 0)),
        ],
        out_shape=[
            jax.ShapeDtypeStruct((total, _D), jnp.float32),
            jax.ShapeDtypeStruct((total, 1), jnp.int32),
            jax.ShapeDtypeStruct((grid_n, 1, 1), jnp.float32),
            jax.ShapeDtypeStruct((grid_n, 1, _K), jnp.float32),
        ],
        compiler_params=pltpu.CompilerParams(
            dimension_semantics=("parallel",)),
    )(x_flat, W, W + W, colsum)

    loss, perp = pl.pallas_call(
        functools.partial(_fin_body, total_tokens=total),
        in_specs=[
            pl.BlockSpec((grid_n, 1, 1), lambda: (0, 0, 0)),
            pl.BlockSpec((grid_n, 1, _K), lambda: (0, 0, 0)),
        ],
        out_specs=[
            pl.BlockSpec((1, 1), lambda: (0, 0)),
            pl.BlockSpec((1, 1), lambda: (0, 0)),
        ],
        out_shape=[
            jax.ShapeDtypeStruct((1, 1), jnp.float32),
            jax.ShapeDtypeStruct((1, 1), jnp.float32),
        ],
    )(esqs, counts_p)

    quantized_st = qst.reshape(orig_shape)
    encoding_indices = idx2.reshape(total)
    return (quantized_st, loss[0, 0], perp[0, 0], encoding_indices)


# speedup vs baseline: 1.1111x; 1.1111x over previous
import functools

import jax
import jax.numpy as jnp
from jax.experimental import pallas as pl
from jax.experimental.pallas import tpu as pltpu

_K = 1024
_D = 64
_TB = 1024
_COMMIT = 0.25


def _colsum_body(wt_ref, cs_ref):
    sq = wt_ref[...] * wt_ref[...]
    acc = sq[0:8, :]
    for j in range(1, 8):
        acc = acc + sq[8 * j:8 * j + 8, :]
    a4 = acc[0:4, :] + acc[4:8, :]
    a2 = a4[0:2, :] + a4[2:4, :]
    cs_ref[...] = a2[0:1, :] + a2[1:2, :]


def _vq_body(x_ref, w_ref, w2_ref, cs_ref, qst_ref, idx_ref, dmin_ref,
             counts_ref):
    xb = x_ref[...]
    w = w_ref[...]
    sq = xb * xb
    r64 = jax.lax.broadcasted_iota(jnp.int32, (_D, 8), 0)
    c8 = jax.lax.broadcasted_iota(jnp.int32, (_D, 8), 1)
    sel = (jax.lax.rem(r64, 8) == c8).astype(jnp.float32)
    p = jax.lax.dot_general(sq, sel, (((1,), (0,)), ((), ())),
                            preferred_element_type=jnp.float32)
    a4 = p[:, 0:4] + p[:, 4:8]
    a2 = a4[:, 0:2] + a4[:, 2:4]
    rowsum = a2[:, 0:1] + a2[:, 1:2]
    colsum = cs_ref[...]
    mm2 = jax.lax.dot_general(xb, w2_ref[...], (((1,), (1,)), ((), ())),
                              preferred_element_type=jnp.float32)
    d = (rowsum + colsum) - mm2
    d = jnp.maximum(d, 0.0)
    dmin = jnp.min(d, axis=1, keepdims=True)
    iota = jax.lax.broadcasted_iota(jnp.int32, (_TB, _K), 1)
    idx = jnp.min(jnp.where(d == dmin, iota, _K), axis=1).astype(jnp.int32)

    onehot = (iota == idx[:, None]).astype(jnp.float32)
    q = jax.lax.dot_general(onehot, w, (((1,), (0,)), ((), ())),
                            preferred_element_type=jnp.float32)
    qst_ref[...] = xb + (q - xb)
    idx_ref[...] = idx[:, None]
    dmin_ref[...] = dmin
    ones_row = jnp.ones((1, _TB), jnp.float32)
    counts_ref[...] = jax.lax.dot_general(
        ones_row, onehot, (((1,), (0,)), ((), ())),
        preferred_element_type=jnp.float32)[None]


def _fin_body(dmin_ref, cp_ref, loss_ref, perp_ref, *, total_tokens):
    ssum = jnp.sum(dmin_ref[...])
    loss = ssum / (total_tokens * _D)
    loss_ref[...] = jnp.full((1, 1), loss + _COMMIT * loss, jnp.float32)
    counts = jnp.sum(cp_ref[...], axis=0)
    p = counts * (1.0 / total_tokens)
    ent = jnp.sum(p * jnp.log(p + 1e-10))
    perp_ref[...] = jnp.full((1, 1), jnp.exp(-ent), jnp.float32)


@jax.jit
def kernel(x, W):
    orig_shape = x.shape
    x_flat = x.reshape(-1, _D)
    total = x_flat.shape[0]
    grid_n = total // _TB

    colsum = pl.pallas_call(
        _colsum_body,
        in_specs=[pl.BlockSpec((_D, _K), lambda: (0, 0))],
        out_specs=pl.BlockSpec((1, _K), lambda: (0, 0)),
        out_shape=jax.ShapeDtypeStruct((1, _K), jnp.float32),
    )(W.T)

    qst, idx2, dminv, counts_p = pl.pallas_call(
        _vq_body,
        grid=(grid_n,),
        in_specs=[
            pl.BlockSpec((_TB, _D), lambda i: (i, 0)),
            pl.BlockSpec((_K, _D), lambda i: (0, 0)),
            pl.BlockSpec((_K, _D), lambda i: (0, 0)),
            pl.BlockSpec((1, _K), lambda i: (0, 0)),
        ],
        out_specs=[
            pl.BlockSpec((_TB, _D), lambda i: (i, 0)),
            pl.BlockSpec((_TB, 1), lambda i: (i, 0)),
            pl.BlockSpec((_TB, 1), lambda i: (i, 0)),
            pl.BlockSpec((1, 1, _K), lambda i: (i, 0, 0)),
        ],
        out_shape=[
            jax.ShapeDtypeStruct((total, _D), jnp.float32),
            jax.ShapeDtypeStruct((total, 1), jnp.int32),
            jax.ShapeDtypeStruct((total, 1), jnp.float32),
            jax.ShapeDtypeStruct((grid_n, 1, _K), jnp.float32),
        ],
        compiler_params=pltpu.CompilerParams(
            dimension_semantics=("parallel",)),
    )(x_flat, W, W + W, colsum)

    loss, perp = pl.pallas_call(
        functools.partial(_fin_body, total_tokens=total),
        in_specs=[
            pl.BlockSpec((total, 1), lambda: (0, 0)),
            pl.BlockSpec((grid_n, 1, _K), lambda: (0, 0, 0)),
        ],
        out_specs=[
            pl.BlockSpec((1, 1), lambda: (0, 0)),
            pl.BlockSpec((1, 1), lambda: (0, 0)),
        ],
        out_shape=[
            jax.ShapeDtypeStruct((1, 1), jnp.float32),
            jax.ShapeDtypeStruct((1, 1), jnp.float32),
        ],
    )(dminv, counts_p)

    quantized_st = qst.reshape(orig_shape)
    encoding_indices = idx2.reshape(total)
    return (quantized_st, loss[0, 0], perp[0, 0], encoding_indices)
